# baseline (speedup 1.0000x reference)
import functools

import jax
import jax.numpy as jnp
from jax import lax
from jax.experimental import pallas as pl
from jax.experimental.pallas import tpu as pltpu
from jax.experimental.pallas import tpu_sc as plsc

E = 64
D = 1024
D2 = D // 2
F = 2048
C = 256
EC = E * C
T = 2048
BLOCK = 64
NB = C // BLOCK

NC = 2
NS = 16
L = 16
NW = NC * NS
TPW = T // NW


def _gelu(x):
    return jax.nn.gelu(x, approximate=True)


def _rne16(v):
    b = lax.bitcast_convert_type(v, jnp.int32)
    return (b + 0x7FFF + ((b >> 16) & 1)) >> 16


def _pack_rows(v):
    lo = _rne16(v[:, :D2]) & 0xFFFF
    hi = _rne16(v[:, D2:])
    return lo | (hi << 16)


def _unpack_rows(pk):
    lo = lax.bitcast_convert_type(pk << 16, jnp.float32)
    hi = lax.bitcast_convert_type(pk & jnp.int32(-65536), jnp.float32)
    return jnp.concatenate([lo, hi], axis=1)


def _router_body(x_ref, gw_ref, logits_ref, dest_ref, counts_ref, aux_ref,
                 xpk_ref):
    x = x_ref[...]
    xpk_ref[...] = _pack_rows(x)
    logits = jnp.dot(x, gw_ref[...], preferred_element_type=jnp.float32)
    logits_ref[...] = logits

    m = jnp.max(logits, axis=1, keepdims=True)
    eids = lax.broadcasted_iota(jnp.int32, (T, E), 1)
    idx = jnp.min(jnp.where(logits == m, eids, E), axis=1, keepdims=True)

    oh = (eids == idx).astype(jnp.int32)
    cs = oh
    sh = 1
    while sh < T:
        cs = cs + jnp.concatenate(
            [jnp.zeros((sh, E), jnp.int32), cs[: T - sh]], axis=0)
        sh *= 2
    pos = jnp.sum((cs - 1) * oh, axis=1, keepdims=True)
    counts_ref[...] = jnp.sum(oh, axis=0, keepdims=True)
    dest_ref[...] = jnp.where(pos < C, idx * C + pos, EC)

    ex = jnp.exp(logits - m)
    se = jnp.sum(ex, axis=1, keepdims=True)
    probs = ex / se
    acc = jnp.sum(probs, axis=0, keepdims=True)
    freq = jnp.sum(oh, axis=0, keepdims=True).astype(jnp.float32)
    acc_n = acc / jnp.maximum(jnp.sum(jnp.abs(acc)), 1e-12)
    freq_n = freq / jnp.maximum(jnp.sum(jnp.abs(freq)), 1e-12)
    switch_loss = E * jnp.sum(acc_n * freq_n)
    lse = jnp.log(se) + m
    z_loss = jnp.mean(lse * lse)
    aux_ref[...] = jnp.reshape(switch_loss + 0.1 * z_loss, (1, 1))


_router = pl.pallas_call(
    _router_body,
    out_shape=(
        jax.ShapeDtypeStruct((T, E), jnp.float32),
        jax.ShapeDtypeStruct((T, 1), jnp.int32),
        jax.ShapeDtypeStruct((1, E), jnp.int32),
        jax.ShapeDtypeStruct((1, 1), jnp.float32),
        jax.ShapeDtypeStruct((T, D2), jnp.int32),
    ),
)


def _dispatch_body(xpk_hbm, dest_hbm, buf_hbm, idx_v, rows_v, sem):
    wid = lax.axis_index("s") * NC + lax.axis_index("c")
    base = wid * TPW
    pltpu.sync_copy(xpk_hbm.at[pl.ds(base, TPW)], rows_v)
    pltpu.sync_copy(dest_hbm.at[pl.ds(base, TPW)], idx_v)
    pltpu.async_copy(rows_v, buf_hbm.at[idx_v], sem).wait()


@functools.cache
def _get_dispatch():
    return functools.partial(
        pl.kernel,
        out_type=jax.ShapeDtypeStruct((EC + C, D2), jnp.int32),
        mesh=plsc.VectorSubcoreMesh(
            core_axis_name="c", subcore_axis_name="s",
            num_cores=NC, num_subcores=NS),
        scratch_types=[
            pltpu.VMEM((TPW,), jnp.int32),
            pltpu.VMEM((TPW, D2), jnp.int32),
            pltpu.SemaphoreType.DMA,
        ],
    )(_dispatch_body)


_ZERO_BLK = EC // BLOCK
_TRASH_BLK = _ZERO_BLK + 1


def _ffn_step(s, cts):
    e = jnp.minimum(s // 4, E - 1)
    r = s % 4
    k = jnp.where(r == 3, 0, r + 1)
    cnt = jnp.minimum(cts[e], C)
    active = jnp.logical_and(k * BLOCK < cnt, s < 4 * E)
    return e, r, k, cnt, active


def _ffn_body(counts_ref, buf_ref, w1_ref, w2_ref, eout_ref):
    s = pl.program_id(0)
    e, r, k, cnt, active = _ffn_step(s, counts_ref)

    @pl.when(active)
    def _():
        xs = _unpack_rows(buf_ref[...])
        h = _gelu(jnp.dot(xs.astype(jnp.bfloat16),
                          w1_ref[0].astype(jnp.bfloat16),
                          preferred_element_type=jnp.float32))
        o = jnp.dot(h.astype(jnp.bfloat16), w2_ref[0].astype(jnp.bfloat16),
                    preferred_element_type=jnp.float32)
        eout_ref[...] = _pack_rows(o)

    @pl.when(s == 4 * E)
    def _():
        eout_ref[...] = jnp.zeros((BLOCK, D2), jnp.int32)


def _buf_idx(s, cts):
    e, r, k, cnt, active = _ffn_step(s, cts)
    idx = jnp.where(r == 3, 4 * e,
                    jnp.where(active, 4 * e + k, jnp.maximum(4 * e - 4, 0)))
    return (idx, 0)


def _eout_idx(s, cts):
    e, r, k, cnt, active = _ffn_step(s, cts)
    idx = jnp.where(s == 4 * E, _ZERO_BLK,
                    jnp.where(r == 3, 4 * e,
                              jnp.where(active, 4 * e + k, _TRASH_BLK)))
    return (idx, 0)


_ffn = pl.pallas_call(
    _ffn_body,
    grid_spec=pltpu.PrefetchScalarGridSpec(
        num_scalar_prefetch=1,
        grid=(4 * E + 1,),
        in_specs=[
            pl.BlockSpec((BLOCK, D2), _buf_idx),
            pl.BlockSpec((1, D, F), lambda s, cts: (jnp.minimum(s // 4, E - 1), 0, 0)),
            pl.BlockSpec((1, F, D), lambda s, cts: (jnp.minimum(s // 4, E - 1), 0, 0)),
        ],
        out_specs=pl.BlockSpec((BLOCK, D2), _eout_idx),
    ),
    out_shape=jax.ShapeDtypeStruct((EC + C, D2), jnp.int32),
)


def _combine_body(eout_hbm, dest_hbm, y_hbm, idx_v, rows_v, sem):
    wid = lax.axis_index("s") * NC + lax.axis_index("c")
    base = wid * TPW
    pltpu.sync_copy(dest_hbm.at[pl.ds(base, TPW)], idx_v)
    pltpu.async_copy(eout_hbm.at[idx_v], rows_v, sem).wait()
    pltpu.sync_copy(rows_v, y_hbm.at[pl.ds(base, TPW)])


@functools.cache
def _get_combine():
    return functools.partial(
        pl.kernel,
        out_type=jax.ShapeDtypeStruct((T, D2), jnp.int32),
        mesh=plsc.VectorSubcoreMesh(
            core_axis_name="c", subcore_axis_name="s",
            num_cores=NC, num_subcores=NS),
        scratch_types=[
            pltpu.VMEM((TPW,), jnp.int32),
            pltpu.VMEM((TPW, D2), jnp.int32),
            pltpu.SemaphoreType.DMA,
        ],
    )(_combine_body)


def kernel(hidden_states, gate_w, w1, w2):
    B_, S_, D_ = hidden_states.shape
    x = hidden_states.reshape(T, D)
    logits, dest2, counts, aux, xpk = _router(x, gate_w)
    dest = dest2.reshape(T)
    buf = _get_dispatch()(xpk, dest)
    eout = _ffn(counts.reshape(E), buf, w1, w2)
    ypk = _get_combine()(eout, dest)
    y = _unpack_rows(ypk)
    return y.reshape(B_, S_, D_), logits, aux[0, 0]


# speedup vs baseline: 1.1750x; 1.1750x over previous
import functools

import jax
import jax.numpy as jnp
from jax import lax
from jax.experimental import pallas as pl
from jax.experimental.pallas import tpu as pltpu
from jax.experimental.pallas import tpu_sc as plsc

E = 64
D = 1024
D2 = D // 2
F = 2048
C = 256
EC = E * C
T = 2048
BLOCK = 64
NB = C // BLOCK

NC = 2
NS = 16
L = 16
NW = NC * NS
TPW = T // NW


def _gelu(x):
    return jax.nn.gelu(x, approximate=True)


def _rne16(v):
    b = lax.bitcast_convert_type(v, jnp.int32)
    return (b + 0x7FFF + ((b >> 16) & 1)) >> 16


def _pack_rows(v):
    lo = _rne16(v[:, :D2]) & 0xFFFF
    hi = _rne16(v[:, D2:])
    return lo | (hi << 16)


def _unpack_rows(pk):
    lo = lax.bitcast_convert_type(pk << 16, jnp.float32)
    hi = lax.bitcast_convert_type(pk & jnp.int32(-65536), jnp.float32)
    return jnp.concatenate([lo, hi], axis=1)


def _router_body(x_ref, gw_ref, logits_ref, dest_ref, counts_ref, aux_ref,
                 xpk_ref, d64_ref):
    x = x_ref[...]
    xpk_ref[...] = _pack_rows(x)
    logits = jnp.dot(x, gw_ref[...], preferred_element_type=jnp.float32)
    logits_ref[...] = logits

    m = jnp.max(logits, axis=1, keepdims=True)
    eids = lax.broadcasted_iota(jnp.int32, (T, E), 1)
    idx = jnp.min(jnp.where(logits == m, eids, E), axis=1, keepdims=True)

    oh = (eids == idx).astype(jnp.int32)
    cs = oh
    sh = 1
    while sh < T:
        cs = cs + jnp.concatenate(
            [jnp.zeros((sh, E), jnp.int32), cs[: T - sh]], axis=0)
        sh *= 2
    pos = jnp.sum((cs - 1) * oh, axis=1, keepdims=True)
    counts_ref[...] = jnp.sum(oh, axis=0, keepdims=True)
    dest_ref[...] = jnp.where(pos < C, idx * C + pos, EC)
    d64_ref[...] = idx * BLOCK + jnp.minimum(pos, BLOCK - 1)

    ex = jnp.exp(logits - m)
    se = jnp.sum(ex, axis=1, keepdims=True)
    probs = ex / se
    acc = jnp.sum(probs, axis=0, keepdims=True)
    freq = jnp.sum(oh, axis=0, keepdims=True).astype(jnp.float32)
    acc_n = acc / jnp.maximum(jnp.sum(jnp.abs(acc)), 1e-12)
    freq_n = freq / jnp.maximum(jnp.sum(jnp.abs(freq)), 1e-12)
    switch_loss = E * jnp.sum(acc_n * freq_n)
    lse = jnp.log(se) + m
    z_loss = jnp.mean(lse * lse)
    aux_ref[...] = jnp.reshape(switch_loss + 0.1 * z_loss, (1, 1))


_router = pl.pallas_call(
    _router_body,
    out_shape=(
        jax.ShapeDtypeStruct((T, E), jnp.float32),
        jax.ShapeDtypeStruct((T, 1), jnp.int32),
        jax.ShapeDtypeStruct((1, E), jnp.int32),
        jax.ShapeDtypeStruct((1, 1), jnp.float32),
        jax.ShapeDtypeStruct((T, D2), jnp.int32),
        jax.ShapeDtypeStruct((T, 1), jnp.int32),
    ),
)


def _dispatch_body(xpk_hbm, dest_hbm, buf_hbm, idx_v, rows_v, sem):
    wid = lax.axis_index("s") * NC + lax.axis_index("c")
    base = wid * TPW
    pltpu.sync_copy(xpk_hbm.at[pl.ds(base, TPW)], rows_v)
    pltpu.sync_copy(dest_hbm.at[pl.ds(base, TPW)], idx_v)
    pltpu.async_copy(rows_v, buf_hbm.at[idx_v], sem).wait()


@functools.cache
def _get_dispatch(nrows):
    return functools.partial(
        pl.kernel,
        out_type=jax.ShapeDtypeStruct((nrows, D2), jnp.int32),
        mesh=plsc.VectorSubcoreMesh(
            core_axis_name="c", subcore_axis_name="s",
            num_cores=NC, num_subcores=NS),
        scratch_types=[
            pltpu.VMEM((TPW,), jnp.int32),
            pltpu.VMEM((TPW, D2), jnp.int32),
            pltpu.SemaphoreType.DMA,
        ],
    )(_dispatch_body)


def _ffn_body(counts_ref, buf_ref, w1_ref, w2_ref, eout_ref):
    e = pl.program_id(0)
    cnt = jnp.where(e < E, jnp.minimum(counts_ref[0, jnp.minimum(e, E - 1)], C), 0)
    for k in range(NB):
        @pl.when(k * BLOCK < cnt)
        def _(k=k):
            xs = _unpack_rows(buf_ref[pl.ds(k * BLOCK, BLOCK), :])
            h = _gelu(jnp.dot(xs.astype(jnp.bfloat16),
                              w1_ref[0].astype(jnp.bfloat16),
                              preferred_element_type=jnp.float32))
            o = jnp.dot(h.astype(jnp.bfloat16), w2_ref[0].astype(jnp.bfloat16),
                        preferred_element_type=jnp.float32)
            eout_ref[pl.ds(k * BLOCK, BLOCK), :] = _pack_rows(o)

    @pl.when(e == E)
    def _():
        eout_ref[...] = jnp.zeros((C, D2), jnp.int32)


_ffn = pl.pallas_call(
    _ffn_body,
    grid=(E + 1,),
    in_specs=[
        pl.BlockSpec(memory_space=pltpu.SMEM),
        pl.BlockSpec((C, D2), lambda e: (e, 0)),
        pl.BlockSpec((1, D, F), lambda e: (jnp.minimum(e, E - 1), 0, 0)),
        pl.BlockSpec((1, F, D), lambda e: (jnp.minimum(e, E - 1), 0, 0)),
    ],
    out_specs=pl.BlockSpec((C, D2), lambda e: (e, 0)),
    out_shape=jax.ShapeDtypeStruct((EC + C, D2), jnp.int32),
)


def _ffn64_body(counts_ref, buf_ref, w1_ref, w2_ref, eout_ref):
    e = pl.program_id(0)
    cnt = counts_ref[0, e]

    @pl.when(cnt > 0)
    def _():
        xs = _unpack_rows(buf_ref[...])
        h = _gelu(jnp.dot(xs.astype(jnp.bfloat16),
                          w1_ref[0].astype(jnp.bfloat16),
                          preferred_element_type=jnp.float32))
        o = jnp.dot(h.astype(jnp.bfloat16), w2_ref[0].astype(jnp.bfloat16),
                    preferred_element_type=jnp.float32)
        eout_ref[...] = _pack_rows(o)


_ffn64 = pl.pallas_call(
    _ffn64_body,
    grid=(E,),
    in_specs=[
        pl.BlockSpec(memory_space=pltpu.SMEM),
        pl.BlockSpec((BLOCK, D2), lambda e: (e, 0)),
        pl.BlockSpec((1, D, F), lambda e: (e, 0, 0)),
        pl.BlockSpec((1, F, D), lambda e: (e, 0, 0)),
    ],
    out_specs=pl.BlockSpec((BLOCK, D2), lambda e: (e, 0)),
    out_shape=jax.ShapeDtypeStruct((E * BLOCK, D2), jnp.int32),
)


def _combine_body(eout_hbm, dest_hbm, y_hbm, idx_v, rows_v, sem):
    wid = lax.axis_index("s") * NC + lax.axis_index("c")
    base = wid * TPW
    pltpu.sync_copy(dest_hbm.at[pl.ds(base, TPW)], idx_v)
    pltpu.async_copy(eout_hbm.at[idx_v], rows_v, sem).wait()
    pltpu.sync_copy(rows_v, y_hbm.at[pl.ds(base, TPW)])


@functools.cache
def _get_combine():
    return functools.partial(
        pl.kernel,
        out_type=jax.ShapeDtypeStruct((T, D2), jnp.int32),
        mesh=plsc.VectorSubcoreMesh(
            core_axis_name="c", subcore_axis_name="s",
            num_cores=NC, num_subcores=NS),
        scratch_types=[
            pltpu.VMEM((TPW,), jnp.int32),
            pltpu.VMEM((TPW, D2), jnp.int32),
            pltpu.SemaphoreType.DMA,
        ],
    )(_combine_body)


def _slow_path(xpk, dest2, d64, counts, w1, w2):
    dest = dest2.reshape(T)
    buf = _get_dispatch(EC + C)(xpk, dest)
    eout = _ffn(counts, buf, w1, w2)
    return _get_combine()(eout, dest)


def _fast_path(xpk, dest2, d64, counts, w1, w2):
    dest = d64.reshape(T)
    buf = _get_dispatch(E * BLOCK)(xpk, dest)
    eout = _ffn64(counts, buf, w1, w2)
    return _get_combine()(eout, dest)


def kernel(hidden_states, gate_w, w1, w2):
    B_, S_, D_ = hidden_states.shape
    x = hidden_states.reshape(T, D)
    logits, dest2, counts, aux, xpk, d64 = _router(x, gate_w)
    overflow = jnp.any(counts > BLOCK)
    ypk = lax.cond(overflow, _slow_path, _fast_path,
                   xpk, dest2, d64, counts, w1, w2)
    y = _unpack_rows(ypk)
    return y.reshape(B_, S_, D_), logits, aux[0, 0]
